# Initial kernel scaffold; baseline (speedup 1.0000x reference)
import jax
import jax.numpy as jnp
from jax.experimental import pallas as pl


def kernel(x, x_batch, tgt_y, tgt_edge_index, tgt_edge_type, tgt_y_batch, embed_table, gcn1_W_self, gcn1_W_msg, gcn1_W_ctx, gcn1_a_gate, gcn2_W_self, gcn2_W_msg, gcn2_W_ctx, gcn2_a_gate, gcn3_W_self, gcn3_W_msg, gcn3_W_ctx, gcn3_a_gate, lin_z_W, lin_z_b, lin_g_W, lin_g_b):
    raise NotImplementedError("write your pallas kernel here")


# speedup vs baseline: 3.8934x; 3.8934x over previous
import functools

import jax
import jax.numpy as jnp
from jax import lax
from jax.experimental import pallas as pl
from jax.experimental.pallas import tpu as pltpu
from jax.experimental.pallas import tpu_sc as plsc

N = 10000
NPAD = 10240
NROW = 10240
E = 160000
EPAD = 163840
ER = EPAD // 128
EW = EPAD // 32
LR = EW // 128
PER = EPAD // 16
PLR = PER // 128
ZR = 10240 // 16
F = 256
V = 512
B = 16
NB = 10
BN = N // NB
f32 = jnp.float32
i32 = jnp.int32


def _mesh():
    return plsc.VectorSubcoreMesh(core_axis_name="c", subcore_axis_name="s")


def _tc_seg(x, xb3):
    def body(x_ref, b_ref, ssum_ref, scnt_ref):
        i = pl.program_id(0)
        ids = b_ref[0, 0, :]
        oh = (ids[:, None] == lax.broadcasted_iota(i32, (BN, B), 1)).astype(f32)
        ps = lax.dot_general(oh, x_ref[...], (((0,), (0,)), ((), ())))
        pc = jnp.broadcast_to(jnp.sum(oh, axis=0)[:, None], (B, F))

        @pl.when(i == 0)
        def _():
            ssum_ref[...] = ps
            scnt_ref[...] = pc

        @pl.when(i > 0)
        def _():
            ssum_ref[...] = ssum_ref[...] + ps
            scnt_ref[...] = scnt_ref[...] + pc

    return pl.pallas_call(
        body,
        grid=(NB,),
        in_specs=[pl.BlockSpec((BN, F), lambda i: (i, 0)),
                  pl.BlockSpec((1, 1, BN), lambda i: (i, 0, 0))],
        out_specs=[pl.BlockSpec((B, F), lambda i: (0, 0)),
                   pl.BlockSpec((B, F), lambda i: (0, 0))],
        out_shape=[jax.ShapeDtypeStruct((B, F), f32)] * 2,
    )(x, xb3)


def _tc_small(ssum, scnt, wctx3, emb, wz, bz2):
    def body(ss_ref, sc_ref, wc_ref, emb_ref, wz_ref, bz_ref, zt_ref, ctx_ref):
        mean = ss_ref[...] / jnp.maximum(sc_ref[...], 1.0)
        for l in range(3):
            ctx_ref[l] = jnp.dot(mean, wc_ref[l])
        zt_ref[...] = jnp.dot(emb_ref[...], wz_ref[...]) + bz_ref[...]

    return pl.pallas_call(
        body,
        out_shape=[jax.ShapeDtypeStruct((V, V), f32),
                   jax.ShapeDtypeStruct((3, B, F), f32)],
    )(ssum, scnt, wctx3, emb, wz, bz2)


def _tc_p(yh, wre):
    def body(y_ref, w_ref, out_ref, hs_ref):
        yfull = jnp.concatenate([y_ref[0], y_ref[1]], axis=-1)
        r = jnp.dot(yfull, w_ref[0])
        for t in range(5):
            out_ref[0, :, t, :] = r[:, t * 128:(t + 1) * 128]
        hs_ref[0] = r[:, :128]

    return pl.pallas_call(
        body,
        grid=(2, NB),
        in_specs=[pl.BlockSpec((2, BN, 128), lambda c, i: (0, i, 0)),
                  pl.BlockSpec((1, F, 640), lambda c, i: (c, 0, 0))],
        out_specs=[pl.BlockSpec((1, BN, 5, 128), lambda c, i: (c, i, 0, 0)),
                   pl.BlockSpec((1, BN, 128), lambda c, i: (c, i, 0))],
        out_shape=[jax.ShapeDtypeStruct((2, N, 5, 128), f32),
                   jax.ShapeDtypeStruct((2, N, 128), f32)],
    )(yh, wre)


def _tc_comb(hs, aggh, deg, yb3, ctx, ag2, final):
    def body(p_ref, a_ref, d_ref, b_ref, ctx_ref, ag_ref, y_ref, al_ref):
        dc = jnp.maximum(d_ref[0, :, :1] + d_ref[1, :, :1], 1.0)
        s0 = p_ref[0] + a_ref[0] / dc
        s1 = p_ref[1] + a_ref[1] / dc
        ids = b_ref[0, 0, :]
        oh = (ids[:, None] == lax.broadcasted_iota(i32, (BN, B), 1)).astype(f32)
        cy = jnp.dot(oh, ctx_ref[...])
        cy0 = cy[:, :128]
        cy1 = cy[:, 128:]
        ag = ag_ref[0]
        g = (lax.dot_general(s0 + cy0, ag[:128], (((1,), (0,)), ((), ()))) +
             lax.dot_general(s1 + cy1, ag[128:], (((1,), (0,)), ((), ()))))
        alpha = jax.nn.sigmoid(g)
        o0 = jax.nn.relu(s0 + alpha[:, None] * cy0)
        o1 = jax.nn.relu(s1 + alpha[:, None] * cy1)
        if final:
            y_ref[:, :128] = o0
            y_ref[:, 128:] = o1
        else:
            y_ref[0] = o0
            y_ref[1] = o1
        al_ref[0, 0, :] = alpha

    if final:
        y_spec = pl.BlockSpec((BN, F), lambda i: (i, 0))
        y_shape = jax.ShapeDtypeStruct((N, F), f32)
    else:
        y_spec = pl.BlockSpec((2, BN, 128), lambda i: (0, i, 0))
        y_shape = jax.ShapeDtypeStruct((2, N, 128), f32)
    return pl.pallas_call(
        body,
        grid=(NB,),
        in_specs=[pl.BlockSpec((2, BN, 128), lambda i: (0, i, 0)),
                  pl.BlockSpec((2, BN, 128), lambda i: (0, i, 0)),
                  pl.BlockSpec((2, BN, 128), lambda i: (0, i, 0)),
                  pl.BlockSpec((1, 1, BN), lambda i: (i, 0, 0)),
                  pl.BlockSpec((B, F), lambda i: (0, 0)),
                  pl.BlockSpec((1, F), lambda i: (0, 0))],
        out_specs=[y_spec, pl.BlockSpec((1, 1, BN), lambda i: (i, 0, 0))],
        out_shape=[y_shape, jax.ShapeDtypeStruct((NB, 1, BN), f32)],
    )(hs, aggh, deg, yb3, ctx, ag2)


def _tc_out(y, wz, bz2, wgcat, bg128):
    def body(y_ref, wz_ref, bz_ref, wg_ref, bg_ref, ys_ref, yt_ref):
        yb = y_ref[...]
        ys_ref[...] = jnp.dot(yb, wz_ref[...]) + bz_ref[...]
        yt_ref[...] = jnp.dot(yb, wg_ref[...]) + bg_ref[...]

    return pl.pallas_call(
        body,
        grid=(NB,),
        in_specs=[pl.BlockSpec((BN, F), lambda i: (i, 0)),
                  pl.BlockSpec((F, V), lambda i: (0, 0)),
                  pl.BlockSpec((1, V), lambda i: (0, 0)),
                  pl.BlockSpec((F, 128), lambda i: (0, 0)),
                  pl.BlockSpec((1, 128), lambda i: (0, 0))],
        out_specs=[pl.BlockSpec((BN, V), lambda i: (i, 0)),
                   pl.BlockSpec((BN, 128), lambda i: (i, 0))],
        out_shape=[jax.ShapeDtypeStruct((N, V), f32),
                   jax.ShapeDtypeStruct((N, 128), f32)],
    )(y, wz, bz2, wgcat, bg128)


def _tc_gidx(src2d, typ2d):
    def body(s_ref, t_ref, g_ref):
        g2 = s_ref[...] * 5 + t_ref[...] + 1
        g_ref[0] = g2
        g_ref[1] = g2 + 5 * N

    return pl.pallas_call(
        body,
        out_shape=jax.ShapeDtypeStruct((2, ER, 128), i32),
    )(src2d, typ2d)


def _sc_y0(emb2, typad):
    per = NPAD // 16

    @functools.partial(
        pl.kernel,
        out_type=jax.ShapeDtypeStruct((2, NPAD, 128), f32),
        mesh=_mesh(),
        scratch_types=[pltpu.VMEM((per,), i32),
                       pltpu.VMEM((per,), i32),
                       pltpu.VMEM((per, 128), f32),
                       pltpu.SemaphoreType.DMA])
    def k(emb_hbm, ty_hbm, out_hbm, tyv, idxv, rows, sem):
        c = lax.axis_index("c")
        s = lax.axis_index("s")
        base = s * per
        pltpu.sync_copy(ty_hbm.at[pl.ds(base, per)], tyv)

        def gb(i, _):
            v = tyv[pl.ds(i * 16, 16)]
            idxv[pl.ds(i * 16, 16)] = v * 2 + c
            return 0

        lax.fori_loop(0, per // 16, gb, 0)
        pltpu.async_copy(emb_hbm.at[idxv], rows, sem).wait()
        pltpu.sync_copy(rows, out_hbm.at[c, pl.ds(base, per)])

    return k(emb2, typad)


def _sc_embeds(zt, ty):
    @functools.partial(
        pl.kernel,
        out_type=jax.ShapeDtypeStruct((N, V), f32),
        mesh=_mesh(),
        scratch_types=[pltpu.VMEM((320,), i32),
                       pltpu.VMEM((160, V), f32),
                       pltpu.SemaphoreType.DMA])
    def k(zt_hbm, ty_hbm, out_hbm, tyv, rows, sem):
        c = lax.axis_index("c")
        s = lax.axis_index("s")
        wid = s * 2 + c
        base = wid * 320

        @pl.when(wid < 31)
        def _():
            pltpu.sync_copy(ty_hbm.at[pl.ds(base, 320)], tyv)
            for j in range(2):
                pltpu.async_copy(zt_hbm.at[tyv.at[pl.ds(j * 160, 160)]],
                                 rows, sem).wait()
                pltpu.sync_copy(rows, out_hbm.at[pl.ds(base + j * 160, 160)])

        @pl.when(wid == 31)
        def _():
            pltpu.sync_copy(ty_hbm.at[pl.ds(base, 80)], tyv.at[pl.ds(0, 80)])
            pltpu.async_copy(zt_hbm.at[tyv.at[pl.ds(0, 80)]],
                             rows.at[pl.ds(0, 80)], sem).wait()
            pltpu.sync_copy(rows.at[pl.ds(0, 80)], out_hbm.at[pl.ds(base, 80)])

    return k(zt, ty)


def _sc_deg(dst2d, ones128, zeros640):
    @functools.partial(
        pl.kernel,
        out_type=jax.ShapeDtypeStruct((2, NROW, 128), f32),
        mesh=_mesh(),
        scratch_types=[pltpu.VMEM((LR, 128), i32),
                       pltpu.VMEM((16, 128), f32),
                       pltpu.VMEM_SHARED((NROW, 128), f32),
                       pltpu.SemaphoreType.DMA])
    def k(d_hbm, ones_hbm, z_hbm, deg_hbm, dstv, onesv, deg_s, sem):
        c = lax.axis_index("c")
        s = lax.axis_index("s")
        wid = s * 2 + c
        pltpu.sync_copy(d_hbm.at[pl.ds(wid * LR, LR)], dstv)
        pltpu.sync_copy(ones_hbm, onesv)
        pltpu.sync_copy(z_hbm, deg_s.at[pl.ds(s * ZR, ZR)])
        plsc.subcore_barrier()

        def db(j, _):
            for t in range(8):
                idxv = dstv[j, pl.ds(t * 16, 16)]
                pltpu.sync_copy(onesv, deg_s.at[idxv], add=True)
            return 0

        lax.fori_loop(0, LR, db, 0)
        plsc.subcore_barrier()
        pltpu.sync_copy(deg_s.at[pl.ds(s * ZR, ZR)],
                        deg_hbm.at[c, pl.ds(s * ZR, ZR)])

    return k(dst2d, ones128, zeros640)


def _sc_agg(pflat, gcs, dst2d, zeros640):
    @functools.partial(
        pl.kernel,
        out_type=jax.ShapeDtypeStruct((2, NROW, 128), f32),
        mesh=_mesh(),
        scratch_types=[pltpu.VMEM((PLR, 128), i32),
                       pltpu.VMEM((PLR, 128), i32),
                       pltpu.VMEM((128, 128), f32),
                       pltpu.VMEM_SHARED((NROW, 128), f32),
                       pltpu.SemaphoreType.DMA])
    def k(p_hbm, g_hbm, d_hbm, z_hbm, out_hbm, gv, dv, rows, agg_s, sem):
        c = lax.axis_index("c")
        s = lax.axis_index("s")
        pltpu.sync_copy(g_hbm.at[c, pl.ds(s * PLR, PLR)], gv)
        pltpu.sync_copy(d_hbm.at[pl.ds(s * PLR, PLR)], dv)
        pltpu.sync_copy(z_hbm, agg_s.at[pl.ds(s * ZR, ZR)])
        plsc.subcore_barrier()

        def lb(j, _):
            pltpu.async_copy(p_hbm.at[gv.at[j]], rows, sem).wait()
            for t in range(8):
                idxv = dv[j, pl.ds(t * 16, 16)]
                pltpu.sync_copy(rows.at[pl.ds(t * 16, 16)],
                                agg_s.at[idxv], add=True)
            return 0

        lax.fori_loop(0, PLR, lb, 0)
        plsc.subcore_barrier()
        pltpu.sync_copy(agg_s.at[pl.ds(s * ZR, ZR)],
                        out_hbm.at[c, pl.ds(s * ZR, ZR)])

    return k(pflat, gcs, dst2d, zeros640)


def _sc_edge(ytab, src2d, dst2d):
    @functools.partial(
        pl.kernel,
        out_type=jax.ShapeDtypeStruct((EPAD, 16), f32),
        mesh=_mesh(),
        scratch_types=[pltpu.VMEM((LR, 128), i32),
                       pltpu.VMEM((LR, 128), i32),
                       pltpu.VMEM((128, 128), f32),
                       pltpu.VMEM((128, 128), f32),
                       pltpu.VMEM((128, 16), f32),
                       pltpu.SemaphoreType.DMA,
                       pltpu.SemaphoreType.DMA])
    def k(yt_hbm, src_hbm, dst_hbm, out_hbm,
          srcv, dstv, rowss, rowsd, outv, sema, semb):
        c = lax.axis_index("c")
        s = lax.axis_index("s")
        wid = s * 2 + c
        rbase = wid * LR
        pltpu.sync_copy(src_hbm.at[pl.ds(rbase, LR)], srcv)
        pltpu.sync_copy(dst_hbm.at[pl.ds(rbase, LR)], dstv)

        def cb(i, _):
            r = i // 8
            o = (i % 8) * 16
            dstv[r, pl.ds(o, 16)] = jnp.minimum(dstv[r, pl.ds(o, 16)], N - 1)
            return 0

        lax.fori_loop(0, LR * 8, cb, 0)

        for j in range(LR):
            ca = pltpu.async_copy(yt_hbm.at[srcv.at[j]], rowss, sema)
            cb2 = pltpu.async_copy(yt_hbm.at[dstv.at[j]], rowsd, semb)
            ca.wait()
            cb2.wait()

            def ab(r, _):
                outv[r, :] = rowss[r, pl.ds(0, 16)] + rowsd[r, pl.ds(16, 16)]
                return 0

            lax.fori_loop(0, 128, ab, 0)
            pltpu.sync_copy(outv, out_hbm.at[pl.ds(wid * EW + j * 128, 128)])

    return k(ytab, src2d, dst2d)


def _wcat(w_self, w_msg):
    wc = jnp.concatenate([w_self[None], w_msg], axis=0)
    wre = wc.reshape(5, F, 2, 128)
    wre = jnp.transpose(wre, (2, 1, 0, 3))
    return wre.reshape(2, F, 640)


def kernel(x, x_batch, tgt_y, tgt_edge_index, tgt_edge_type, tgt_y_batch,
           embed_table,
           gcn1_W_self, gcn1_W_msg, gcn1_W_ctx, gcn1_a_gate,
           gcn2_W_self, gcn2_W_msg, gcn2_W_ctx, gcn2_a_gate,
           gcn3_W_self, gcn3_W_msg, gcn3_W_ctx, gcn3_a_gate,
           lin_z_W, lin_z_b, lin_g_W, lin_g_b):
    src = tgt_edge_index[0].astype(i32)
    dst = tgt_edge_index[1].astype(i32)
    src2d = jnp.pad(src, (0, EPAD - E)).reshape(ER, 128)
    dst2d = jnp.pad(dst, (0, EPAD - E), constant_values=N).reshape(ER, 128)
    typ2d = jnp.pad(tgt_edge_type.astype(i32), (0, EPAD - E)).reshape(ER, 128)
    ty = tgt_y.astype(i32)
    typad = jnp.pad(ty, (0, NPAD - N))
    xb3 = x_batch.astype(i32).reshape(NB, 1, BN)
    yb3 = tgt_y_batch.astype(i32).reshape(NB, 1, BN)
    emb2 = embed_table.reshape(2 * V, 128)
    wctx3 = jnp.stack([gcn1_W_ctx, gcn2_W_ctx, gcn3_W_ctx])
    wre = [_wcat(gcn1_W_self, gcn1_W_msg), _wcat(gcn2_W_self, gcn2_W_msg),
           _wcat(gcn3_W_self, gcn3_W_msg)]
    ag2 = [gcn1_a_gate.reshape(1, F), gcn2_a_gate.reshape(1, F),
           gcn3_a_gate.reshape(1, F)]
    bz2 = lin_z_b.reshape(1, V)
    wgcat = jnp.concatenate(
        [lin_g_W[:F], jnp.zeros((F, 11), f32),
         lin_g_W[F:], jnp.zeros((F, 107), f32)], axis=1)
    bg128 = jnp.pad(lin_g_b, (0, 128 - 5)).reshape(1, 128)
    ones128 = jnp.ones((16, 128), f32)
    zeros640 = jnp.zeros((ZR, 128), f32)

    ssum, scnt = _tc_seg(x, xb3)
    zt, ctxs = _tc_small(ssum, scnt, wctx3, embed_table, lin_z_W, bz2)
    embeds = _sc_embeds(zt, ty)
    y0h = _sc_y0(emb2, typad)
    gcs = _tc_gidx(src2d, typ2d)
    deg = _sc_deg(dst2d, ones128, zeros640)

    yh = y0h
    alphas = []
    for l in range(3):
        p, hs = _tc_p(yh, wre[l])
        aggh = _sc_agg(p.reshape(10 * N, 128), gcs, dst2d, zeros640)
        final = l == 2
        yh, al = _tc_comb(hs, aggh, deg, yb3, ctxs[l], ag2[l], final)
        alphas.append(al.reshape(N))
    y = yh

    y_score, ytab = _tc_out(y, lin_z_W, bz2, wgcat, bg128)
    esum = _sc_edge(ytab, src2d, dst2d)
    y_edge = esum[:E, :5]

    return (y, tgt_edge_index, tgt_edge_type, y_score, y_edge, embeds,
            alphas[0], alphas[1], alphas[2])


# speedup vs baseline: 4.5978x; 1.1809x over previous
import functools

import jax
import jax.numpy as jnp
from jax import lax
from jax.experimental import pallas as pl
from jax.experimental.pallas import tpu as pltpu
from jax.experimental.pallas import tpu_sc as plsc

N = 10000
NPAD = 10240
NROW = 10240
E = 160000
EPAD = 163840
ER = EPAD // 128
EW = EPAD // 32
LR = EW // 128
PER = EPAD // 16
PLR = PER // 128
ZR = 10240 // 16
F = 256
V = 512
B = 16
NB = 10
BN = N // NB
f32 = jnp.float32
i32 = jnp.int32


def _mesh():
    return plsc.VectorSubcoreMesh(core_axis_name="c", subcore_axis_name="s")


def _tc_seg(x, xb3):
    def body(x_ref, b_ref, ssum_ref, scnt_ref):
        i = pl.program_id(0)
        ids = b_ref[0, 0, :]
        oh = (ids[:, None] == lax.broadcasted_iota(i32, (BN, B), 1)).astype(f32)
        ps = lax.dot_general(oh, x_ref[...], (((0,), (0,)), ((), ())))
        pc = jnp.broadcast_to(jnp.sum(oh, axis=0)[:, None], (B, F))

        @pl.when(i == 0)
        def _():
            ssum_ref[...] = ps
            scnt_ref[...] = pc

        @pl.when(i > 0)
        def _():
            ssum_ref[...] = ssum_ref[...] + ps
            scnt_ref[...] = scnt_ref[...] + pc

    return pl.pallas_call(
        body,
        grid=(NB,),
        in_specs=[pl.BlockSpec((BN, F), lambda i: (i, 0)),
                  pl.BlockSpec((1, 1, BN), lambda i: (i, 0, 0))],
        out_specs=[pl.BlockSpec((B, F), lambda i: (0, 0)),
                   pl.BlockSpec((B, F), lambda i: (0, 0))],
        out_shape=[jax.ShapeDtypeStruct((B, F), f32)] * 2,
    )(x, xb3)


def _tc_small(ssum, scnt, wctx3, emb, wz, bz2):
    def body(ss_ref, sc_ref, wc_ref, emb_ref, wz_ref, bz_ref, zt_ref, ctx_ref):
        mean = ss_ref[...] / jnp.maximum(sc_ref[...], 1.0)
        for l in range(3):
            ctx_ref[l] = jnp.dot(mean, wc_ref[l])
        zt_ref[...] = jnp.dot(emb_ref[...], wz_ref[...]) + bz_ref[...]

    return pl.pallas_call(
        body,
        out_shape=[jax.ShapeDtypeStruct((V, V), f32),
                   jax.ShapeDtypeStruct((3, B, F), f32)],
    )(ssum, scnt, wctx3, emb, wz, bz2)


def _tc_p(yh, wre):
    def body(y_ref, w_ref, out_ref, hs_ref):
        yfull = jnp.concatenate([y_ref[0], y_ref[1]], axis=-1)
        r = jnp.dot(yfull, w_ref[0])
        for t in range(5):
            out_ref[0, :, t, :] = r[:, t * 128:(t + 1) * 128]
        hs_ref[0] = r[:, :128]

    return pl.pallas_call(
        body,
        grid=(2, NB),
        in_specs=[pl.BlockSpec((2, BN, 128), lambda c, i: (0, i, 0)),
                  pl.BlockSpec((1, F, 640), lambda c, i: (c, 0, 0))],
        out_specs=[pl.BlockSpec((1, BN, 5, 128), lambda c, i: (c, i, 0, 0)),
                   pl.BlockSpec((1, BN, 128), lambda c, i: (c, i, 0))],
        out_shape=[jax.ShapeDtypeStruct((2, N, 5, 128), f32),
                   jax.ShapeDtypeStruct((2, N, 128), f32)],
    )(yh, wre)


def _tc_comb(hs, aggh, deg, yb3, ctx, ag2, final):
    def body(p_ref, a_ref, d_ref, b_ref, ctx_ref, ag_ref, y_ref, al_ref):
        dc = jnp.maximum(d_ref[0, :, :1] + d_ref[1, :, :1], 1.0)
        s0 = p_ref[0] + a_ref[0] / dc
        s1 = p_ref[1] + a_ref[1] / dc
        ids = b_ref[0, 0, :]
        oh = (ids[:, None] == lax.broadcasted_iota(i32, (BN, B), 1)).astype(f32)
        cy = jnp.dot(oh, ctx_ref[...])
        cy0 = cy[:, :128]
        cy1 = cy[:, 128:]
        ag = ag_ref[0]
        g = (lax.dot_general(s0 + cy0, ag[:128], (((1,), (0,)), ((), ()))) +
             lax.dot_general(s1 + cy1, ag[128:], (((1,), (0,)), ((), ()))))
        alpha = jax.nn.sigmoid(g)
        o0 = jax.nn.relu(s0 + alpha[:, None] * cy0)
        o1 = jax.nn.relu(s1 + alpha[:, None] * cy1)
        if final:
            y_ref[:, :128] = o0
            y_ref[:, 128:] = o1
        else:
            y_ref[0] = o0
            y_ref[1] = o1
        al_ref[0, 0, :] = alpha

    if final:
        y_spec = pl.BlockSpec((BN, F), lambda i: (i, 0))
        y_shape = jax.ShapeDtypeStruct((N, F), f32)
    else:
        y_spec = pl.BlockSpec((2, BN, 128), lambda i: (0, i, 0))
        y_shape = jax.ShapeDtypeStruct((2, N, 128), f32)
    return pl.pallas_call(
        body,
        grid=(NB,),
        in_specs=[pl.BlockSpec((2, BN, 128), lambda i: (0, i, 0)),
                  pl.BlockSpec((2, BN, 128), lambda i: (0, i, 0)),
                  pl.BlockSpec((2, BN, 128), lambda i: (0, i, 0)),
                  pl.BlockSpec((1, 1, BN), lambda i: (i, 0, 0)),
                  pl.BlockSpec((B, F), lambda i: (0, 0)),
                  pl.BlockSpec((1, F), lambda i: (0, 0))],
        out_specs=[y_spec, pl.BlockSpec((1, 1, BN), lambda i: (i, 0, 0))],
        out_shape=[y_shape, jax.ShapeDtypeStruct((NB, 1, BN), f32)],
    )(hs, aggh, deg, yb3, ctx, ag2)


def _tc_out(y, wz, bz2, wgcat, bg128):
    def body(y_ref, wz_ref, bz_ref, wg_ref, bg_ref, ys_ref, yt_ref):
        yb = y_ref[...]
        ys_ref[...] = jnp.dot(yb, wz_ref[...]) + bz_ref[...]
        yt_ref[...] = jnp.dot(yb, wg_ref[...]) + bg_ref[...]

    return pl.pallas_call(
        body,
        grid=(NB,),
        in_specs=[pl.BlockSpec((BN, F), lambda i: (i, 0)),
                  pl.BlockSpec((F, V), lambda i: (0, 0)),
                  pl.BlockSpec((1, V), lambda i: (0, 0)),
                  pl.BlockSpec((F, 128), lambda i: (0, 0)),
                  pl.BlockSpec((1, 128), lambda i: (0, 0))],
        out_specs=[pl.BlockSpec((BN, V), lambda i: (i, 0)),
                   pl.BlockSpec((BN, 128), lambda i: (i, 0))],
        out_shape=[jax.ShapeDtypeStruct((N, V), f32),
                   jax.ShapeDtypeStruct((N, 128), f32)],
    )(y, wz, bz2, wgcat, bg128)


def _tc_gidx(src2d, typ2d):
    def body(s_ref, t_ref, g_ref):
        g2 = s_ref[...] * 5 + t_ref[...] + 1
        g_ref[0] = g2
        g_ref[1] = g2 + 5 * N

    return pl.pallas_call(
        body,
        out_shape=jax.ShapeDtypeStruct((2, ER, 128), i32),
    )(src2d, typ2d)


def _sc_y0(emb2, typad):
    per = NPAD // 16

    @functools.partial(
        pl.kernel,
        out_type=jax.ShapeDtypeStruct((2, NPAD, 128), f32),
        mesh=_mesh(),
        scratch_types=[pltpu.VMEM((per,), i32),
                       pltpu.VMEM((per,), i32),
                       pltpu.VMEM((per, 128), f32),
                       pltpu.SemaphoreType.DMA])
    def k(emb_hbm, ty_hbm, out_hbm, tyv, idxv, rows, sem):
        c = lax.axis_index("c")
        s = lax.axis_index("s")
        base = s * per
        pltpu.sync_copy(ty_hbm.at[pl.ds(base, per)], tyv)

        def gb(i, _):
            v = tyv[pl.ds(i * 16, 16)]
            idxv[pl.ds(i * 16, 16)] = v * 2 + c
            return 0

        lax.fori_loop(0, per // 16, gb, 0)
        pltpu.async_copy(emb_hbm.at[idxv], rows, sem).wait()
        pltpu.sync_copy(rows, out_hbm.at[c, pl.ds(base, per)])

    return k(emb2, typad)


def _sc_embeds(zt, ty):
    @functools.partial(
        pl.kernel,
        out_type=jax.ShapeDtypeStruct((N, V), f32),
        mesh=_mesh(),
        scratch_types=[pltpu.VMEM((320,), i32),
                       pltpu.VMEM((160, V), f32),
                       pltpu.SemaphoreType.DMA])
    def k(zt_hbm, ty_hbm, out_hbm, tyv, rows, sem):
        c = lax.axis_index("c")
        s = lax.axis_index("s")
        wid = s * 2 + c
        base = wid * 320

        @pl.when(wid < 31)
        def _():
            pltpu.sync_copy(ty_hbm.at[pl.ds(base, 320)], tyv)
            for j in range(2):
                pltpu.async_copy(zt_hbm.at[tyv.at[pl.ds(j * 160, 160)]],
                                 rows, sem).wait()
                pltpu.sync_copy(rows, out_hbm.at[pl.ds(base + j * 160, 160)])

        @pl.when(wid == 31)
        def _():
            pltpu.sync_copy(ty_hbm.at[pl.ds(base, 80)], tyv.at[pl.ds(0, 80)])
            pltpu.async_copy(zt_hbm.at[tyv.at[pl.ds(0, 80)]],
                             rows.at[pl.ds(0, 80)], sem).wait()
            pltpu.sync_copy(rows.at[pl.ds(0, 80)], out_hbm.at[pl.ds(base, 80)])

    return k(zt, ty)


def _sc_deg(dst2d, ones128, zeros640):
    @functools.partial(
        pl.kernel,
        out_type=jax.ShapeDtypeStruct((2, NROW, 128), f32),
        mesh=_mesh(),
        scratch_types=[pltpu.VMEM((LR, 128), i32),
                       pltpu.VMEM((128, 128), f32),
                       pltpu.VMEM_SHARED((NROW, 128), f32),
                       pltpu.SemaphoreType.DMA])
    def k(d_hbm, ones_hbm, z_hbm, deg_hbm, dstv, onesv, deg_s, sem):
        c = lax.axis_index("c")
        s = lax.axis_index("s")
        wid = s * 2 + c
        pltpu.sync_copy(d_hbm.at[pl.ds(wid * LR, LR)], dstv)
        pltpu.sync_copy(ones_hbm, onesv)
        pltpu.sync_copy(z_hbm, deg_s.at[pl.ds(s * ZR, ZR)])
        plsc.subcore_barrier()

        def db(j, _):
            pltpu.sync_copy(onesv, deg_s.at[dstv.at[j]], add=True)
            return 0

        lax.fori_loop(0, LR, db, 0)
        plsc.subcore_barrier()
        pltpu.sync_copy(deg_s.at[pl.ds(s * ZR, ZR)],
                        deg_hbm.at[c, pl.ds(s * ZR, ZR)])

    return k(dst2d, ones128, zeros640)


def _sc_agg(pflat, gcs, dst2d, zeros640):
    G = 8
    K = 2
    NG = PLR // G

    @functools.partial(
        pl.kernel,
        out_type=jax.ShapeDtypeStruct((2, NROW, 128), f32),
        mesh=_mesh(),
        scratch_types=[pltpu.VMEM((G, 128), i32),
                       pltpu.VMEM((G, 128), i32),
                       pltpu.VMEM((K * 128, 128), f32),
                       pltpu.VMEM_SHARED((NROW, 128), f32),
                       pltpu.SemaphoreType.DMA])
    def k(p_hbm, g_hbm, d_hbm, z_hbm, out_hbm, gvb, dvb, rows, agg_s, sem):
        c = lax.axis_index("c")
        s = lax.axis_index("s")
        pltpu.sync_copy(z_hbm, agg_s.at[pl.ds(s * ZR, ZR)])
        plsc.subcore_barrier()

        def group(g, _):
            base = s * PLR + g * G
            pltpu.sync_copy(g_hbm.at[c, pl.ds(base, G)], gvb)
            pltpu.sync_copy(d_hbm.at[pl.ds(base, G)], dvb)
            for k0 in range(K):
                pltpu.async_copy(p_hbm.at[gvb.at[k0]],
                                 rows.at[pl.ds(k0 * 128, 128)], sem)
            for j in range(G):
                b = (j % K) * 128
                pltpu.make_async_copy(p_hbm.at[gvb.at[j]],
                                      rows.at[pl.ds(b, 128)], sem).wait()
                pltpu.sync_copy(rows.at[pl.ds(b, 128)],
                                agg_s.at[dvb.at[j]], add=True)
                if j + K < G:
                    pltpu.async_copy(p_hbm.at[gvb.at[j + K]],
                                     rows.at[pl.ds(b, 128)], sem)
            return 0

        lax.fori_loop(0, NG, group, 0)
        plsc.subcore_barrier()
        pltpu.sync_copy(agg_s.at[pl.ds(s * ZR, ZR)],
                        out_hbm.at[c, pl.ds(s * ZR, ZR)])

    return k(pflat, gcs, dst2d, zeros640)


def _sc_edge(ytab, src2d, dst2d):
    @functools.partial(
        pl.kernel,
        out_type=jax.ShapeDtypeStruct((EPAD, 16), f32),
        mesh=_mesh(),
        scratch_types=[pltpu.VMEM((LR, 128), i32),
                       pltpu.VMEM((LR, 128), i32),
                       pltpu.VMEM((128, 128), f32),
                       pltpu.VMEM((128, 128), f32),
                       pltpu.VMEM((128, 16), f32),
                       pltpu.SemaphoreType.DMA,
                       pltpu.SemaphoreType.DMA])
    def k(yt_hbm, src_hbm, dst_hbm, out_hbm,
          srcv, dstv, rowss, rowsd, outv, sema, semb):
        c = lax.axis_index("c")
        s = lax.axis_index("s")
        wid = s * 2 + c
        rbase = wid * LR
        pltpu.sync_copy(src_hbm.at[pl.ds(rbase, LR)], srcv)
        pltpu.sync_copy(dst_hbm.at[pl.ds(rbase, LR)], dstv)

        def cb(i, _):
            r = i // 8
            o = (i % 8) * 16
            dstv[r, pl.ds(o, 16)] = jnp.minimum(dstv[r, pl.ds(o, 16)], N - 1)
            return 0

        lax.fori_loop(0, LR * 8, cb, 0)

        for j in range(LR):
            ca = pltpu.async_copy(yt_hbm.at[srcv.at[j]], rowss, sema)
            cb2 = pltpu.async_copy(yt_hbm.at[dstv.at[j]], rowsd, semb)
            ca.wait()
            cb2.wait()

            def ab(r, _):
                outv[r, :] = rowss[r, pl.ds(0, 16)] + rowsd[r, pl.ds(16, 16)]
                return 0

            lax.fori_loop(0, 128, ab, 0)
            pltpu.sync_copy(outv, out_hbm.at[pl.ds(wid * EW + j * 128, 128)])

    return k(ytab, src2d, dst2d)


def _wcat(w_self, w_msg):
    wc = jnp.concatenate([w_self[None], w_msg], axis=0)
    wre = wc.reshape(5, F, 2, 128)
    wre = jnp.transpose(wre, (2, 1, 0, 3))
    return wre.reshape(2, F, 640)


def kernel(x, x_batch, tgt_y, tgt_edge_index, tgt_edge_type, tgt_y_batch,
           embed_table,
           gcn1_W_self, gcn1_W_msg, gcn1_W_ctx, gcn1_a_gate,
           gcn2_W_self, gcn2_W_msg, gcn2_W_ctx, gcn2_a_gate,
           gcn3_W_self, gcn3_W_msg, gcn3_W_ctx, gcn3_a_gate,
           lin_z_W, lin_z_b, lin_g_W, lin_g_b):
    src = tgt_edge_index[0].astype(i32)
    dst = tgt_edge_index[1].astype(i32)
    src2d = jnp.pad(src, (0, EPAD - E)).reshape(ER, 128)
    dst2d = jnp.pad(dst, (0, EPAD - E), constant_values=N).reshape(ER, 128)
    typ2d = jnp.pad(tgt_edge_type.astype(i32), (0, EPAD - E)).reshape(ER, 128)
    ty = tgt_y.astype(i32)
    typad = jnp.pad(ty, (0, NPAD - N))
    xb3 = x_batch.astype(i32).reshape(NB, 1, BN)
    yb3 = tgt_y_batch.astype(i32).reshape(NB, 1, BN)
    emb2 = embed_table.reshape(2 * V, 128)
    wctx3 = jnp.stack([gcn1_W_ctx, gcn2_W_ctx, gcn3_W_ctx])
    wre = [_wcat(gcn1_W_self, gcn1_W_msg), _wcat(gcn2_W_self, gcn2_W_msg),
           _wcat(gcn3_W_self, gcn3_W_msg)]
    ag2 = [gcn1_a_gate.reshape(1, F), gcn2_a_gate.reshape(1, F),
           gcn3_a_gate.reshape(1, F)]
    bz2 = lin_z_b.reshape(1, V)
    wgcat = jnp.concatenate(
        [lin_g_W[:F], jnp.zeros((F, 11), f32),
         lin_g_W[F:], jnp.zeros((F, 107), f32)], axis=1)
    bg128 = jnp.pad(lin_g_b, (0, 128 - 5)).reshape(1, 128)
    ones128 = jnp.ones((128, 128), f32)
    zeros640 = jnp.zeros((ZR, 128), f32)

    ssum, scnt = _tc_seg(x, xb3)
    zt, ctxs = _tc_small(ssum, scnt, wctx3, embed_table, lin_z_W, bz2)
    embeds = _sc_embeds(zt, ty)
    y0h = _sc_y0(emb2, typad)
    gcs = _tc_gidx(src2d, typ2d)
    deg = _sc_deg(dst2d, ones128, zeros640)

    yh = y0h
    alphas = []
    for l in range(3):
        p, hs = _tc_p(yh, wre[l])
        aggh = _sc_agg(p.reshape(10 * N, 128), gcs, dst2d, zeros640)
        final = l == 2
        yh, al = _tc_comb(hs, aggh, deg, yb3, ctxs[l], ag2[l], final)
        alphas.append(al.reshape(N))
    y = yh

    y_score, ytab = _tc_out(y, lin_z_W, bz2, wgcat, bg128)
    esum = _sc_edge(ytab, src2d, dst2d)
    y_edge = esum[:E, :5]

    return (y, tgt_edge_index, tgt_edge_type, y_score, y_edge, embeds,
            alphas[0], alphas[1], alphas[2])


# speedup vs baseline: 4.6831x; 1.0185x over previous
import functools

import jax
import jax.numpy as jnp
from jax import lax
from jax.experimental import pallas as pl
from jax.experimental.pallas import tpu as pltpu
from jax.experimental.pallas import tpu_sc as plsc

N = 10000
NPAD = 10240
NROW = 10240
E = 160000
EPAD = 163840
ER = EPAD // 128
EW = EPAD // 32
LR = EW // 128
PER = EPAD // 16
PLR = PER // 128
ZR = 10240 // 16
F = 256
V = 512
B = 16
NB = 10
BN = N // NB
f32 = jnp.float32
i32 = jnp.int32


def _mesh():
    return plsc.VectorSubcoreMesh(core_axis_name="c", subcore_axis_name="s")


def _tc_seg(x, xb3):
    def body(x_ref, b_ref, ssum_ref, scnt_ref):
        i = pl.program_id(0)
        ids = b_ref[0, 0, :]
        oh = (ids[:, None] == lax.broadcasted_iota(i32, (BN, B), 1)).astype(f32)
        ps = lax.dot_general(oh, x_ref[...], (((0,), (0,)), ((), ())))
        pc = jnp.broadcast_to(jnp.sum(oh, axis=0)[:, None], (B, F))

        @pl.when(i == 0)
        def _():
            ssum_ref[...] = ps
            scnt_ref[...] = pc

        @pl.when(i > 0)
        def _():
            ssum_ref[...] = ssum_ref[...] + ps
            scnt_ref[...] = scnt_ref[...] + pc

    return pl.pallas_call(
        body,
        grid=(NB,),
        in_specs=[pl.BlockSpec((BN, F), lambda i: (i, 0)),
                  pl.BlockSpec((1, 1, BN), lambda i: (i, 0, 0))],
        out_specs=[pl.BlockSpec((B, F), lambda i: (0, 0)),
                   pl.BlockSpec((B, F), lambda i: (0, 0))],
        out_shape=[jax.ShapeDtypeStruct((B, F), f32)] * 2,
    )(x, xb3)


def _tc_small(ssum, scnt, wctx3, emb, wz, bz2):
    def body(ss_ref, sc_ref, wc_ref, emb_ref, wz_ref, bz_ref, zt_ref, ctx_ref):
        mean = ss_ref[...] / jnp.maximum(sc_ref[...], 1.0)
        for l in range(3):
            ctx_ref[l] = jnp.dot(mean, wc_ref[l])
        zt_ref[...] = jnp.dot(emb_ref[...], wz_ref[...]) + bz_ref[...]

    return pl.pallas_call(
        body,
        out_shape=[jax.ShapeDtypeStruct((V, V), f32),
                   jax.ShapeDtypeStruct((3, B, F), f32)],
    )(ssum, scnt, wctx3, emb, wz, bz2)


def _tc_p(yh, wre):
    def body(y_ref, w_ref, out_ref, hs_ref):
        yfull = jnp.concatenate([y_ref[0], y_ref[1]], axis=-1)
        r = jnp.dot(yfull, w_ref[0])
        for t in range(5):
            out_ref[0, :, t, :] = r[:, t * 128:(t + 1) * 128]
        hs_ref[0] = r[:, :128]

    return pl.pallas_call(
        body,
        grid=(2, NB),
        in_specs=[pl.BlockSpec((2, BN, 128), lambda c, i: (0, i, 0)),
                  pl.BlockSpec((1, F, 640), lambda c, i: (c, 0, 0))],
        out_specs=[pl.BlockSpec((1, BN, 5, 128), lambda c, i: (c, i, 0, 0)),
                   pl.BlockSpec((1, BN, 128), lambda c, i: (c, i, 0))],
        out_shape=[jax.ShapeDtypeStruct((2, N, 5, 128), f32),
                   jax.ShapeDtypeStruct((2, N, 128), f32)],
    )(yh, wre)


def _tc_comb(hs, aggh, deg, yb3, ctx, ag2, final):
    def body(p_ref, a_ref, d_ref, b_ref, ctx_ref, ag_ref, y_ref, al_ref):
        dc = jnp.maximum(d_ref[0, :, :1] + d_ref[1, :, :1], 1.0)
        s0 = p_ref[0] + a_ref[0] / dc
        s1 = p_ref[1] + a_ref[1] / dc
        ids = b_ref[0, 0, :]
        oh = (ids[:, None] == lax.broadcasted_iota(i32, (BN, B), 1)).astype(f32)
        cy = jnp.dot(oh, ctx_ref[...])
        cy0 = cy[:, :128]
        cy1 = cy[:, 128:]
        ag = ag_ref[0]
        g = (lax.dot_general(s0 + cy0, ag[:128], (((1,), (0,)), ((), ()))) +
             lax.dot_general(s1 + cy1, ag[128:], (((1,), (0,)), ((), ()))))
        alpha = jax.nn.sigmoid(g)
        o0 = jax.nn.relu(s0 + alpha[:, None] * cy0)
        o1 = jax.nn.relu(s1 + alpha[:, None] * cy1)
        if final:
            y_ref[:, :128] = o0
            y_ref[:, 128:] = o1
        else:
            y_ref[0] = o0
            y_ref[1] = o1
        al_ref[0, 0, :] = alpha

    if final:
        y_spec = pl.BlockSpec((BN, F), lambda i: (i, 0))
        y_shape = jax.ShapeDtypeStruct((N, F), f32)
    else:
        y_spec = pl.BlockSpec((2, BN, 128), lambda i: (0, i, 0))
        y_shape = jax.ShapeDtypeStruct((2, N, 128), f32)
    return pl.pallas_call(
        body,
        grid=(NB,),
        in_specs=[pl.BlockSpec((2, BN, 128), lambda i: (0, i, 0)),
                  pl.BlockSpec((2, BN, 128), lambda i: (0, i, 0)),
                  pl.BlockSpec((2, BN, 128), lambda i: (0, i, 0)),
                  pl.BlockSpec((1, 1, BN), lambda i: (i, 0, 0)),
                  pl.BlockSpec((B, F), lambda i: (0, 0)),
                  pl.BlockSpec((1, F), lambda i: (0, 0))],
        out_specs=[y_spec, pl.BlockSpec((1, 1, BN), lambda i: (i, 0, 0))],
        out_shape=[y_shape, jax.ShapeDtypeStruct((NB, 1, BN), f32)],
    )(hs, aggh, deg, yb3, ctx, ag2)


def _tc_out(y, wz, bz2, wgcat, bg128):
    def body(y_ref, wz_ref, bz_ref, wg_ref, bg_ref, ys_ref, yt_ref):
        yb = y_ref[...]
        ys_ref[...] = jnp.dot(yb, wz_ref[...]) + bz_ref[...]
        yt_ref[...] = jnp.dot(yb, wg_ref[...]) + bg_ref[...]

    return pl.pallas_call(
        body,
        grid=(NB,),
        in_specs=[pl.BlockSpec((BN, F), lambda i: (i, 0)),
                  pl.BlockSpec((F, V), lambda i: (0, 0)),
                  pl.BlockSpec((1, V), lambda i: (0, 0)),
                  pl.BlockSpec((F, 128), lambda i: (0, 0)),
                  pl.BlockSpec((1, 128), lambda i: (0, 0))],
        out_specs=[pl.BlockSpec((BN, V), lambda i: (i, 0)),
                   pl.BlockSpec((BN, 128), lambda i: (i, 0))],
        out_shape=[jax.ShapeDtypeStruct((N, V), f32),
                   jax.ShapeDtypeStruct((N, 128), f32)],
    )(y, wz, bz2, wgcat, bg128)


def _tc_gidx(src2d, typ2d, dst2d):
    def body(s_ref, t_ref, d_ref, g_ref, dc_ref):
        g2 = s_ref[...] * 5 + t_ref[...] + 1
        g_ref[0] = g2
        g_ref[1] = g2 + 5 * N
        dc_ref[...] = jnp.minimum(d_ref[...], N - 1)

    return pl.pallas_call(
        body,
        out_shape=[jax.ShapeDtypeStruct((2, ER, 128), i32),
                   jax.ShapeDtypeStruct((ER, 128), i32)],
    )(src2d, typ2d, dst2d)


def _sc_y0(emb2, typad):
    per = NPAD // 16

    @functools.partial(
        pl.kernel,
        out_type=jax.ShapeDtypeStruct((2, NPAD, 128), f32),
        mesh=_mesh(),
        scratch_types=[pltpu.VMEM((per,), i32),
                       pltpu.VMEM((per,), i32),
                       pltpu.VMEM((per, 128), f32),
                       pltpu.SemaphoreType.DMA])
    def k(emb_hbm, ty_hbm, out_hbm, tyv, idxv, rows, sem):
        c = lax.axis_index("c")
        s = lax.axis_index("s")
        base = s * per
        pltpu.sync_copy(ty_hbm.at[pl.ds(base, per)], tyv)

        def gb(i, _):
            v = tyv[pl.ds(i * 16, 16)]
            idxv[pl.ds(i * 16, 16)] = v * 2 + c
            return 0

        lax.fori_loop(0, per // 16, gb, 0)
        pltpu.async_copy(emb_hbm.at[idxv], rows, sem).wait()
        pltpu.sync_copy(rows, out_hbm.at[c, pl.ds(base, per)])

    return k(emb2, typad)


def _sc_embeds(zt, ty):
    @functools.partial(
        pl.kernel,
        out_type=jax.ShapeDtypeStruct((N, V), f32),
        mesh=_mesh(),
        scratch_types=[pltpu.VMEM((320,), i32),
                       pltpu.VMEM((160, V), f32),
                       pltpu.SemaphoreType.DMA])
    def k(zt_hbm, ty_hbm, out_hbm, tyv, rows, sem):
        c = lax.axis_index("c")
        s = lax.axis_index("s")
        wid = s * 2 + c
        base = wid * 320

        @pl.when(wid < 31)
        def _():
            pltpu.sync_copy(ty_hbm.at[pl.ds(base, 320)], tyv)
            for j in range(2):
                pltpu.async_copy(zt_hbm.at[tyv.at[pl.ds(j * 160, 160)]],
                                 rows, sem).wait()
                pltpu.sync_copy(rows, out_hbm.at[pl.ds(base + j * 160, 160)])

        @pl.when(wid == 31)
        def _():
            pltpu.sync_copy(ty_hbm.at[pl.ds(base, 80)], tyv.at[pl.ds(0, 80)])
            pltpu.async_copy(zt_hbm.at[tyv.at[pl.ds(0, 80)]],
                             rows.at[pl.ds(0, 80)], sem).wait()
            pltpu.sync_copy(rows.at[pl.ds(0, 80)], out_hbm.at[pl.ds(base, 80)])

    return k(zt, ty)


def _sc_deg(dst2d, ones128, zeros640):
    @functools.partial(
        pl.kernel,
        out_type=jax.ShapeDtypeStruct((2, NROW, 128), f32),
        mesh=_mesh(),
        scratch_types=[pltpu.VMEM((LR, 128), i32),
                       pltpu.VMEM((128, 128), f32),
                       pltpu.VMEM_SHARED((NROW, 128), f32),
                       pltpu.SemaphoreType.DMA])
    def k(d_hbm, ones_hbm, z_hbm, deg_hbm, dstv, onesv, deg_s, sem):
        c = lax.axis_index("c")
        s = lax.axis_index("s")
        wid = s * 2 + c
        pltpu.sync_copy(d_hbm.at[pl.ds(wid * LR, LR)], dstv)
        pltpu.sync_copy(ones_hbm, onesv)
        pltpu.sync_copy(z_hbm, deg_s.at[pl.ds(s * ZR, ZR)])
        plsc.subcore_barrier()

        def db(j, _):
            pltpu.sync_copy(onesv, deg_s.at[dstv.at[j]], add=True)
            return 0

        lax.fori_loop(0, LR, db, 0)
        plsc.subcore_barrier()
        pltpu.sync_copy(deg_s.at[pl.ds(s * ZR, ZR)],
                        deg_hbm.at[c, pl.ds(s * ZR, ZR)])

    return k(dst2d, ones128, zeros640)


def _sc_agg(pflat, gcs, dst2d, zeros640):
    G = 8
    K = 2
    NG = PLR // G

    @functools.partial(
        pl.kernel,
        out_type=jax.ShapeDtypeStruct((2, NROW, 128), f32),
        mesh=_mesh(),
        scratch_types=[pltpu.VMEM((G, 128), i32),
                       pltpu.VMEM((G, 128), i32),
                       pltpu.VMEM((K * 128, 128), f32),
                       pltpu.VMEM_SHARED((NROW, 128), f32),
                       pltpu.SemaphoreType.DMA])
    def k(p_hbm, g_hbm, d_hbm, z_hbm, out_hbm, gvb, dvb, rows, agg_s, sem):
        c = lax.axis_index("c")
        s = lax.axis_index("s")
        pltpu.sync_copy(z_hbm, agg_s.at[pl.ds(s * ZR, ZR)])
        plsc.subcore_barrier()

        def group(g, _):
            base = s * PLR + g * G
            pltpu.sync_copy(g_hbm.at[c, pl.ds(base, G)], gvb)
            pltpu.sync_copy(d_hbm.at[pl.ds(base, G)], dvb)
            for k0 in range(K):
                pltpu.async_copy(p_hbm.at[gvb.at[k0]],
                                 rows.at[pl.ds(k0 * 128, 128)], sem)
            for j in range(G):
                b = (j % K) * 128
                pltpu.make_async_copy(p_hbm.at[gvb.at[j]],
                                      rows.at[pl.ds(b, 128)], sem).wait()
                pltpu.sync_copy(rows.at[pl.ds(b, 128)],
                                agg_s.at[dvb.at[j]], add=True)
                if j + K < G:
                    pltpu.async_copy(p_hbm.at[gvb.at[j + K]],
                                     rows.at[pl.ds(b, 128)], sem)
            return 0

        lax.fori_loop(0, NG, group, 0)
        plsc.subcore_barrier()
        pltpu.sync_copy(agg_s.at[pl.ds(s * ZR, ZR)],
                        out_hbm.at[c, pl.ds(s * ZR, ZR)])

    return k(pflat, gcs, dst2d, zeros640)


def _sc_edge(ytab, ilist):
    K = 3

    @functools.partial(
        pl.kernel,
        out_type=jax.ShapeDtypeStruct((EPAD, 16), f32),
        mesh=_mesh(),
        scratch_types=[pltpu.VMEM((LR * 256,), i32),
                       pltpu.VMEM((K * 256, 128), f32),
                       pltpu.VMEM((128, 16), f32),
                       pltpu.SemaphoreType.DMA])
    def k(yt_hbm, i_hbm, out_hbm, iv, rows, outv, sem):
        c = lax.axis_index("c")
        s = lax.axis_index("s")
        wid = s * 2 + c
        pltpu.sync_copy(i_hbm.at[pl.ds(wid * LR * 256, LR * 256)], iv)
        for k0 in range(K):
            pltpu.async_copy(yt_hbm.at[iv.at[pl.ds(k0 * 256, 256)]],
                             rows.at[pl.ds(k0 * 256, 256)], sem)

        def ch(j, _):
            b = (j % K) * 256
            pltpu.make_async_copy(yt_hbm.at[iv.at[pl.ds(j * 256, 256)]],
                                  rows.at[pl.ds(b, 256)], sem).wait()

            def ab(r, _):
                outv[r, :] = (rows[b + r, pl.ds(0, 16)] +
                              rows[b + 128 + r, pl.ds(16, 16)])
                return 0

            lax.fori_loop(0, 128, ab, 0)

            @pl.when(j + K < LR)
            def _():
                pltpu.async_copy(yt_hbm.at[iv.at[pl.ds((j + K) * 256, 256)]],
                                 rows.at[pl.ds(b, 256)], sem)

            pltpu.sync_copy(outv, out_hbm.at[pl.ds(wid * EW + j * 128, 128)])
            return 0

        lax.fori_loop(0, LR, ch, 0)

    return k(ytab, ilist)


def _wcat(w_self, w_msg):
    wc = jnp.concatenate([w_self[None], w_msg], axis=0)
    wre = wc.reshape(5, F, 2, 128)
    wre = jnp.transpose(wre, (2, 1, 0, 3))
    return wre.reshape(2, F, 640)


def kernel(x, x_batch, tgt_y, tgt_edge_index, tgt_edge_type, tgt_y_batch,
           embed_table,
           gcn1_W_self, gcn1_W_msg, gcn1_W_ctx, gcn1_a_gate,
           gcn2_W_self, gcn2_W_msg, gcn2_W_ctx, gcn2_a_gate,
           gcn3_W_self, gcn3_W_msg, gcn3_W_ctx, gcn3_a_gate,
           lin_z_W, lin_z_b, lin_g_W, lin_g_b):
    src = tgt_edge_index[0].astype(i32)
    dst = tgt_edge_index[1].astype(i32)
    src2d = jnp.pad(src, (0, EPAD - E)).reshape(ER, 128)
    dst2d = jnp.pad(dst, (0, EPAD - E), constant_values=N).reshape(ER, 128)
    typ2d = jnp.pad(tgt_edge_type.astype(i32), (0, EPAD - E)).reshape(ER, 128)
    ty = tgt_y.astype(i32)
    typad = jnp.pad(ty, (0, NPAD - N))
    xb3 = x_batch.astype(i32).reshape(NB, 1, BN)
    yb3 = tgt_y_batch.astype(i32).reshape(NB, 1, BN)
    emb2 = embed_table.reshape(2 * V, 128)
    wctx3 = jnp.stack([gcn1_W_ctx, gcn2_W_ctx, gcn3_W_ctx])
    wre = [_wcat(gcn1_W_self, gcn1_W_msg), _wcat(gcn2_W_self, gcn2_W_msg),
           _wcat(gcn3_W_self, gcn3_W_msg)]
    ag2 = [gcn1_a_gate.reshape(1, F), gcn2_a_gate.reshape(1, F),
           gcn3_a_gate.reshape(1, F)]
    bz2 = lin_z_b.reshape(1, V)
    wgcat = jnp.concatenate(
        [lin_g_W[:F], jnp.zeros((F, 11), f32),
         lin_g_W[F:], jnp.zeros((F, 107), f32)], axis=1)
    bg128 = jnp.pad(lin_g_b, (0, 128 - 5)).reshape(1, 128)
    ones128 = jnp.ones((128, 128), f32)
    zeros640 = jnp.zeros((ZR, 128), f32)

    ssum, scnt = _tc_seg(x, xb3)
    zt, ctxs = _tc_small(ssum, scnt, wctx3, embed_table, lin_z_W, bz2)
    embeds = _sc_embeds(zt, ty)
    y0h = _sc_y0(emb2, typad)
    gcs, dstc = _tc_gidx(src2d, typ2d, dst2d)
    deg = _sc_deg(dst2d, ones128, zeros640)

    yh = y0h
    alphas = []
    for l in range(3):
        p, hs = _tc_p(yh, wre[l])
        aggh = _sc_agg(p.reshape(10 * N, 128), gcs, dst2d, zeros640)
        final = l == 2
        yh, al = _tc_comb(hs, aggh, deg, yb3, ctxs[l], ag2[l], final)
        alphas.append(al.reshape(N))
    y = yh

    y_score, ytab = _tc_out(y, lin_z_W, bz2, wgcat, bg128)
    ilist = jnp.concatenate([src2d[:, None, :], dstc[:, None, :]],
                            axis=1).reshape(ER * 256)
    esum = _sc_edge(ytab, ilist)
    y_edge = esum[:E, :5]

    return (y, tgt_edge_index, tgt_edge_type, y_score, y_edge, embeds,
            alphas[0], alphas[1], alphas[2])
